# baseline (speedup 1.0000x reference)
import functools

import jax
import jax.numpy as jnp
from jax.experimental import pallas as pl
from jax.experimental.pallas import tpu as pltpu

B, T, D = 2, 2048, 1024
E, H, K = 8, 512, 2
BT = B * T
BM = 128


def _moe_body(x_ref, selT_ref, k_ref, v_ref, rs_ref, o_ref, kb_ref, vb_ref):
    i = pl.program_id(0)

    @pl.when(i == 0)
    def _cast_weights():
        kb_ref[...] = k_ref[...].astype(jnp.bfloat16)
        vb_ref[...] = v_ref[...].astype(jnp.bfloat16)

    x = x_ref[...]
    logits = jnp.dot(x, selT_ref[...], preferred_element_type=jnp.float32)
    p = jax.nn.sigmoid(logits)
    eidx = jax.lax.broadcasted_iota(jnp.int32, (BM, E), 1)
    cnt = jnp.zeros((BM, E), jnp.int32)
    for a in range(E):
        pa = p[:, a : a + 1]
        beats = (pa > p) | ((pa == p) & (a < eidx))
        cnt = cnt + beats.astype(jnp.int32)
    g = jnp.where(cnt < K, p, 0.0)
    denom = jnp.sum(g, axis=1, keepdims=True)
    w = g / jnp.maximum(denom, 1e-9) * rs_ref[0]

    col_e = jax.lax.broadcasted_iota(jnp.int32, (E, E * H), 1) // H
    row_e = jax.lax.broadcasted_iota(jnp.int32, (E, E * H), 0)
    expand = (col_e == row_e).astype(jnp.float32)
    ws = jnp.dot(w, expand, preferred_element_type=jnp.float32)

    xb = x.astype(jnp.bfloat16)
    parts = []
    for j in range(E):
        h = jnp.dot(xb, kb_ref[j], preferred_element_type=jnp.float32)
        hs = jnp.maximum(h, 0.0) * ws[:, j * H : (j + 1) * H]
        parts.append(hs.astype(jnp.bfloat16))
    hall = jnp.concatenate(parts, axis=1)
    o_ref[...] = jnp.dot(hall, vb_ref[...], preferred_element_type=jnp.float32)


@functools.partial(jax.jit, static_argnames=("interpret",))
def _moe(x2d, selT, keys, values2d, route_scale, interpret=False):
    out = pl.pallas_call(
        _moe_body,
        grid=(BT // BM,),
        in_specs=[
            pl.BlockSpec((BM, D), lambda i: (i, 0)),
            pl.BlockSpec((D, E), lambda i: (0, 0)),
            pl.BlockSpec((E, D, H), lambda i: (0, 0, 0)),
            pl.BlockSpec((E * H, D), lambda i: (0, 0)),
            pl.BlockSpec(memory_space=pltpu.SMEM),
        ],
        out_specs=pl.BlockSpec((BM, D), lambda i: (i, 0)),
        out_shape=jax.ShapeDtypeStruct((BT, D), jnp.float32),
        scratch_shapes=[
            pltpu.VMEM((E, D, H), jnp.bfloat16),
            pltpu.VMEM((E * H, D), jnp.bfloat16),
        ],
        interpret=interpret,
    )(x2d, selT, keys, values2d, route_scale)
    return out


def kernel(input, expert_sel, keys, values, route_scale, interpret=False):
    x2d = input.reshape(BT, D)
    selT = expert_sel.T
    values2d = values.reshape(E * H, D)
    out = _moe(x2d, selT, keys, values2d, route_scale, interpret=interpret)
    return out.reshape(B, T, D)


# speedup vs baseline: 1.0361x; 1.0361x over previous
import functools

import jax
import jax.numpy as jnp
from jax.experimental import pallas as pl
from jax.experimental.pallas import tpu as pltpu

B, T, D = 2, 2048, 1024
E, H, K = 8, 512, 2
BT = B * T
BM = 256


def _moe_body(x_ref, selT_ref, k_ref, v_ref, rs_ref, o_ref, kb_ref, vb_ref):
    i = pl.program_id(0)

    @pl.when(i == 0)
    def _cast_weights():
        kb_ref[...] = k_ref[...].astype(jnp.bfloat16)
        vb_ref[...] = v_ref[...].astype(jnp.bfloat16)

    x = x_ref[...]
    logits = jnp.dot(x, selT_ref[...], preferred_element_type=jnp.float32)
    p = jax.nn.sigmoid(logits)
    eidx = jax.lax.broadcasted_iota(jnp.int32, (BM, E), 1)
    cnt = jnp.zeros((BM, E), jnp.int32)
    for a in range(E):
        pa = p[:, a : a + 1]
        beats = (pa > p) | ((pa == p) & (a < eidx))
        cnt = cnt + beats.astype(jnp.int32)
    g = jnp.where(cnt < K, p, 0.0)
    denom = jnp.sum(g, axis=1, keepdims=True)
    w = g / jnp.maximum(denom, 1e-9) * rs_ref[0]

    col_e = jax.lax.broadcasted_iota(jnp.int32, (E, E * H), 1) // H
    row_e = jax.lax.broadcasted_iota(jnp.int32, (E, E * H), 0)
    expand = (col_e == row_e).astype(jnp.float32)
    ws = jnp.dot(w, expand, preferred_element_type=jnp.float32)

    xb = x.astype(jnp.bfloat16)
    parts = []
    for j in range(E):
        h = jnp.dot(xb, kb_ref[j], preferred_element_type=jnp.float32)
        hs = jnp.maximum(h, 0.0) * ws[:, j * H : (j + 1) * H]
        parts.append(hs.astype(jnp.bfloat16))
    hall = jnp.concatenate(parts, axis=1)
    o_ref[...] = jnp.dot(hall, vb_ref[...], preferred_element_type=jnp.float32)


@functools.partial(jax.jit, static_argnames=("interpret",))
def _moe(x2d, selT, keys, values2d, route_scale, interpret=False):
    out = pl.pallas_call(
        _moe_body,
        grid=(BT // BM,),
        in_specs=[
            pl.BlockSpec((BM, D), lambda i: (i, 0)),
            pl.BlockSpec((D, E), lambda i: (0, 0)),
            pl.BlockSpec((E, D, H), lambda i: (0, 0, 0)),
            pl.BlockSpec((E * H, D), lambda i: (0, 0)),
            pl.BlockSpec(memory_space=pltpu.SMEM),
        ],
        out_specs=pl.BlockSpec((BM, D), lambda i: (i, 0)),
        out_shape=jax.ShapeDtypeStruct((BT, D), jnp.float32),
        scratch_shapes=[
            pltpu.VMEM((E, D, H), jnp.bfloat16),
            pltpu.VMEM((E * H, D), jnp.bfloat16),
        ],
        interpret=interpret,
    )(x2d, selT, keys, values2d, route_scale)
    return out


def kernel(input, expert_sel, keys, values, route_scale, interpret=False):
    x2d = input.reshape(BT, D)
    selT = expert_sel.T
    values2d = values.reshape(E * H, D)
    out = _moe(x2d, selT, keys, values2d, route_scale, interpret=interpret)
    return out.reshape(B, T, D)


# speedup vs baseline: 1.2142x; 1.1719x over previous
import functools

import jax
import jax.numpy as jnp
from jax.experimental import pallas as pl
from jax.experimental.pallas import tpu as pltpu

B, T, D = 2, 2048, 1024
E, H, K = 8, 512, 2
BT = B * T
CHUNK = 1024
NC = BT // CHUNK


def _moe_body(x_ref, selT_ref, k_ref, v_ref, rs_ref, o_ref, w_ref):
    j = pl.program_id(0)

    @pl.when(j == 0)
    def _router():
        x = x_ref[...]
        logits = jnp.dot(x, selT_ref[...], preferred_element_type=jnp.float32)
        p = jax.nn.sigmoid(logits)
        eidx = jax.lax.broadcasted_iota(jnp.int32, (BT, E), 1)
        cnt = jnp.zeros((BT, E), jnp.int32)
        for a in range(E):
            pa = p[:, a : a + 1]
            beats = (pa > p) | ((pa == p) & (a < eidx))
            cnt = cnt + beats.astype(jnp.int32)
        g = jnp.where(cnt < K, p, 0.0)
        denom = jnp.sum(g, axis=1, keepdims=True)
        w_ref[...] = g / jnp.maximum(denom, 1e-9) * rs_ref[0]
        o_ref[...] = jnp.zeros((BT, D), jnp.float32)

    kb = k_ref[0]
    vb = v_ref[0]
    eidx = jax.lax.broadcasted_iota(jnp.int32, (CHUNK, E), 1)
    for c in range(NC):
        sl = pl.ds(c * CHUNK, CHUNK)
        xc = x_ref[sl, :]
        h = jnp.dot(xc, kb, preferred_element_type=jnp.float32)
        wc = w_ref[sl, :]
        wj = jnp.sum(jnp.where(eidx == j, wc, 0.0), axis=1, keepdims=True)
        hs = jnp.maximum(h, 0.0) * wj
        o_ref[sl, :] += jnp.dot(hs, vb, preferred_element_type=jnp.float32)


@functools.partial(jax.jit, static_argnames=("interpret",))
def _moe(x2d, selT, keysT, values, route_scale, interpret=False):
    out = pl.pallas_call(
        _moe_body,
        grid=(E,),
        in_specs=[
            pl.BlockSpec((BT, D), lambda j: (0, 0)),
            pl.BlockSpec((D, E), lambda j: (0, 0)),
            pl.BlockSpec((1, D, H), lambda j: (j, 0, 0)),
            pl.BlockSpec((1, H, D), lambda j: (j, 0, 0)),
            pl.BlockSpec(memory_space=pltpu.SMEM),
        ],
        out_specs=pl.BlockSpec((BT, D), lambda j: (0, 0)),
        out_shape=jax.ShapeDtypeStruct((BT, D), jnp.float32),
        scratch_shapes=[
            pltpu.VMEM((BT, E), jnp.float32),
        ],
        interpret=interpret,
    )(x2d, selT, keysT, values, route_scale)
    return out


def kernel(input, expert_sel, keys, values, route_scale, interpret=False):
    x2d = input.reshape(BT, D)
    selT = expert_sel.T
    out = _moe(x2d, selT, keys, values, route_scale, interpret=interpret)
    return out.reshape(B, T, D)
